# baseline (speedup 1.0000x reference)
import functools

import jax
import jax.numpy as jnp
from jax.experimental import pallas as pl
from jax.experimental.pallas import tpu as pltpu

_WIN = 48
_LANE = 128
_RB = 128


def _pos_body(w_ref, x_ref, hl_ref, hr_ref, out_ref, *, K, nrows):
    RB = _RB
    R1 = RB + 1
    f32 = jnp.float32

    pid = pl.program_id(0)

    crow = jax.lax.broadcasted_iota(jnp.int32, (RB + 2, 128), 0)
    grow = pid * RB - 1 + crow
    in_range = (grow >= 0) & (grow < nrows)
    ch = [jnp.where(in_range,
                    jnp.concatenate(
                        [hl_ref[c, 7:8, :], x_ref[c], hr_ref[c, 0:1, :]],
                        axis=0), 0.0)
          for c in range(3)]

    ii = jax.lax.broadcasted_iota(jnp.int32, (128, 128), 0)
    ll = jax.lax.broadcasted_iota(jnp.int32, (128, 128), 1)
    T1a = ((ii >= ll) & (ii <= ll + (_WIN - 1))).astype(f32)
    T1b = ((ii + 128 >= ll) & (ii + 128 <= ll + (_WIN - 1))).astype(f32)
    T2a = ((ii >= ll + (_LANE - _WIN + 1)) & (ii <= ll + _LANE)).astype(f32)
    T2b = ((ii + 128 >= ll + (_LANE - _WIN + 1))
           & (ii + 128 <= ll + _LANE)).astype(f32)

    def dotf(a, t):
        return jnp.dot(a, t, preferred_element_type=f32)

    def win_sum(fa, fb):
        return dotf(fa, T1a) + dotf(fb, T1b)

    cA = [c_[:R1, :] for c_ in ch]
    cB = [c_[1:R1 + 1, :] for c_ in ch]

    s0 = win_sum(cA[0], cB[0])
    s1 = win_sum(cA[1], cB[1])
    s2 = win_sum(cA[2], cB[2])
    Q00 = win_sum(cA[0] * cA[0], cB[0] * cB[0])
    Q11 = win_sum(cA[1] * cA[1], cB[1] * cB[1])
    Q22 = win_sum(cA[2] * cA[2], cB[2] * cB[2])
    Q01 = win_sum(cA[0] * cA[1], cB[0] * cB[1])
    Q02 = win_sum(cA[0] * cA[2], cB[0] * cB[2])
    Q12 = win_sum(cA[1] * cA[2], cB[1] * cB[2])

    wn = f32(_WIN)
    a0 = wn / s0
    a1 = wn / s1
    a2 = wn / s2
    M00 = a0 * a0 * Q00 - wn
    M11 = a1 * a1 * Q11 - wn
    M22 = a2 * a2 * Q22 - wn
    M01 = a0 * a1 * Q01 - wn
    M02 = a0 * a2 * Q02 - wn
    M12 = a1 * a2 * Q12 - wn

    w00 = w_ref[0, 0]
    w01 = w_ref[0, 1]
    w02 = w_ref[0, 2]
    w10 = w_ref[1, 0]
    w11 = w_ref[1, 1]
    w12 = w_ref[1, 2]

    A2 = (w00 * w00 * M00 + w01 * w01 * M11 + w02 * w02 * M22
          + 2.0 * (w00 * w01 * M01 + w00 * w02 * M02 + w01 * w02 * M12))
    B2 = (w10 * w10 * M00 + w11 * w11 * M11 + w12 * w12 * M22
          + 2.0 * (w10 * w11 * M01 + w10 * w12 * M02 + w11 * w12 * M12))
    r = jnp.sqrt(jnp.maximum(A2, 0.0) / B2)

    g0 = w00 + r * w10
    g1 = w01 + r * w11
    g2 = w02 + r * w12
    p0 = g0 * a0
    p1 = g1 * a1
    p2 = g2 * a2
    p3 = g0 + g1 + g2

    kg = grow[:R1, :] * _LANE + jax.lax.broadcasted_iota(
        jnp.int32, (R1, 128), 1)
    valid = (kg >= 0) & (kg < K)
    p0 = jnp.where(valid, p0, 0.0)
    p1 = jnp.where(valid, p1, 0.0)
    p2 = jnp.where(valid, p2, 0.0)
    p3 = jnp.where(valid, p3, 0.0)

    def back_sum(p):
        return dotf(p[:RB, :], T2a) + dotf(p[1:R1, :], T2b)

    P0 = back_sum(p0)
    P1 = back_sum(p1)
    P2 = back_sum(p2)
    P3 = back_sum(p3)

    out_ref[...] = (ch[0][1:RB + 1, :] * P0 + ch[1][1:RB + 1, :] * P1
                    + ch[2][1:RB + 1, :] * P2 - P3)


def kernel(rgbs, W, b):
    del b
    N = rgbs.shape[1]
    K = N - _WIN
    nrows = N // _LANE
    assert nrows * _LANE == N
    G = -(-nrows // _RB)
    rb8 = _RB // 8
    hmax = (nrows - 8) // 8

    x3 = jnp.transpose(rgbs[0]).reshape(3, nrows, _LANE)

    out = pl.pallas_call(
        functools.partial(_pos_body, K=K, nrows=nrows),
        grid=(G,),
        in_specs=[
            pl.BlockSpec(memory_space=pltpu.SMEM),
            pl.BlockSpec((3, _RB, _LANE), lambda g: (0, g, 0)),
            pl.BlockSpec((3, 8, _LANE),
                         lambda g: (0, jnp.maximum(g * rb8 - 1, 0), 0)),
            pl.BlockSpec((3, 8, _LANE),
                         lambda g: (0, jnp.minimum(g * rb8 + rb8, hmax), 0)),
        ],
        out_specs=pl.BlockSpec((_RB, _LANE), lambda g: (g, 0)),
        out_shape=jax.ShapeDtypeStruct((nrows, _LANE), jnp.float32),
        compiler_params=pltpu.CompilerParams(
            dimension_semantics=("parallel",)),
    )(W.astype(jnp.float32), x3, x3, x3)
    return out.reshape(1, N)


# speedup vs baseline: 1.0617x; 1.0617x over previous
import functools

import jax
import jax.numpy as jnp
from jax.experimental import pallas as pl
from jax.experimental.pallas import tpu as pltpu

_WIN = 48
_LANE = 128
_RB = 128


def _pos_body(w_ref, x_ref, hl_ref, hr_ref, out_ref, *, K, nrows):
    RB = _RB
    R1 = RB + 1
    f32 = jnp.float32

    pid = pl.program_id(0)

    crow = jax.lax.broadcasted_iota(jnp.int32, (RB + 2, 128), 0)
    grow = pid * RB - 1 + crow
    in_range = (grow >= 0) & (grow < nrows)
    ch = [jnp.where(in_range,
                    jnp.concatenate(
                        [hl_ref[c, 7:8, :], x_ref[c], hr_ref[c, 0:1, :]],
                        axis=0), 0.0)
          for c in range(3)]

    X2 = [jnp.concatenate([c_[:R1, :], c_[1:R1 + 1, :]], axis=1) for c_ in ch]

    ii = jax.lax.broadcasted_iota(jnp.int32, (256, 128), 0)
    ll = jax.lax.broadcasted_iota(jnp.int32, (256, 128), 1)
    T1 = ((ii >= ll) & (ii <= ll + (_WIN - 1))).astype(f32)
    T2 = ((ii >= ll + (_LANE - _WIN + 1)) & (ii <= ll + _LANE)).astype(f32)

    def win_sum(a):
        return jnp.dot(a, T1, preferred_element_type=f32)

    s0 = win_sum(X2[0])
    s1 = win_sum(X2[1])
    s2 = win_sum(X2[2])
    Q00 = win_sum(X2[0] * X2[0])
    Q11 = win_sum(X2[1] * X2[1])
    Q22 = win_sum(X2[2] * X2[2])
    Q01 = win_sum(X2[0] * X2[1])
    Q02 = win_sum(X2[0] * X2[2])
    Q12 = win_sum(X2[1] * X2[2])

    wn = f32(_WIN)
    a0 = wn / s0
    a1 = wn / s1
    a2 = wn / s2
    M00 = a0 * a0 * Q00 - wn
    M11 = a1 * a1 * Q11 - wn
    M22 = a2 * a2 * Q22 - wn
    M01 = a0 * a1 * Q01 - wn
    M02 = a0 * a2 * Q02 - wn
    M12 = a1 * a2 * Q12 - wn

    w00 = w_ref[0, 0]
    w01 = w_ref[0, 1]
    w02 = w_ref[0, 2]
    w10 = w_ref[1, 0]
    w11 = w_ref[1, 1]
    w12 = w_ref[1, 2]

    A2 = (w00 * w00 * M00 + w01 * w01 * M11 + w02 * w02 * M22
          + 2.0 * (w00 * w01 * M01 + w00 * w02 * M02 + w01 * w02 * M12))
    B2 = (w10 * w10 * M00 + w11 * w11 * M11 + w12 * w12 * M22
          + 2.0 * (w10 * w11 * M01 + w10 * w12 * M02 + w11 * w12 * M12))
    r = jnp.sqrt(jnp.maximum(A2, 0.0) / B2)

    g0 = w00 + r * w10
    g1 = w01 + r * w11
    g2 = w02 + r * w12
    p0 = g0 * a0
    p1 = g1 * a1
    p2 = g2 * a2
    p3 = g0 + g1 + g2

    kg = grow[:R1, :] * _LANE + jax.lax.broadcasted_iota(
        jnp.int32, (R1, 128), 1)
    valid = (kg >= 0) & (kg < K)
    p0 = jnp.where(valid, p0, 0.0)
    p1 = jnp.where(valid, p1, 0.0)
    p2 = jnp.where(valid, p2, 0.0)
    p3 = jnp.where(valid, p3, 0.0)

    def back_sum(p):
        pr = jnp.concatenate([p[:RB, :], p[1:R1, :]], axis=1)
        return jnp.dot(pr, T2, preferred_element_type=f32)

    P0 = back_sum(p0)
    P1 = back_sum(p1)
    P2 = back_sum(p2)
    P3 = back_sum(p3)

    out_ref[...] = (ch[0][1:RB + 1, :] * P0 + ch[1][1:RB + 1, :] * P1
                    + ch[2][1:RB + 1, :] * P2 - P3)


def kernel(rgbs, W, b):
    del b
    N = rgbs.shape[1]
    K = N - _WIN
    nrows = N // _LANE
    assert nrows * _LANE == N
    G = -(-nrows // _RB)
    rb8 = _RB // 8
    hmax = (nrows - 8) // 8

    x3 = jnp.transpose(rgbs[0]).reshape(3, nrows, _LANE)

    out = pl.pallas_call(
        functools.partial(_pos_body, K=K, nrows=nrows),
        grid=(G,),
        in_specs=[
            pl.BlockSpec(memory_space=pltpu.SMEM),
            pl.BlockSpec((3, _RB, _LANE), lambda g: (0, g, 0)),
            pl.BlockSpec((3, 8, _LANE),
                         lambda g: (0, jnp.maximum(g * rb8 - 1, 0), 0)),
            pl.BlockSpec((3, 8, _LANE),
                         lambda g: (0, jnp.minimum(g * rb8 + rb8, hmax), 0)),
        ],
        out_specs=pl.BlockSpec((_RB, _LANE), lambda g: (g, 0)),
        out_shape=jax.ShapeDtypeStruct((nrows, _LANE), jnp.float32),
        compiler_params=pltpu.CompilerParams(
            dimension_semantics=("parallel",)),
    )(W.astype(jnp.float32), x3, x3, x3)
    return out.reshape(1, N)


# speedup vs baseline: 1.2043x; 1.1344x over previous
import functools

import jax
import jax.numpy as jnp
from jax.experimental import pallas as pl
from jax.experimental.pallas import tpu as pltpu

_WIN = 48
_LANE = 128
_RB = 128


def _pos_body(scal_ref, x_ref, hl_ref, hr_ref, out_ref, *, K, nrows):
    RB = _RB
    R1 = RB + 1
    f32 = jnp.float32

    hl = hl_ref[...]
    hr = hr_ref[...]

    pid = pl.program_id(0)

    crow = jax.lax.broadcasted_iota(jnp.int32, (RB + 2, 128), 0)
    in_range = (pid * RB - 1 + crow) < nrows
    ch = [jnp.where(in_range,
                    jnp.concatenate([hl[:, c, :], x_ref[c], hr[:, c, :]],
                                    axis=0), 0.0)
          for c in range(3)]

    X2 = [jnp.concatenate([c_[:R1, :], c_[1:R1 + 1, :]], axis=1) for c_ in ch]

    ii = jax.lax.broadcasted_iota(jnp.int32, (256, 128), 0)
    ll = jax.lax.broadcasted_iota(jnp.int32, (256, 128), 1)
    T1 = ((ii >= ll) & (ii <= ll + (_WIN - 1))).astype(f32)
    T2 = ((ii >= ll + (_LANE - _WIN + 1)) & (ii <= ll + _LANE)).astype(f32)

    def win_sum(a):
        return jnp.dot(a, T1, preferred_element_type=f32)

    s0 = win_sum(X2[0])
    s1 = win_sum(X2[1])
    s2 = win_sum(X2[2])
    Q00 = win_sum(X2[0] * X2[0])
    Q11 = win_sum(X2[1] * X2[1])
    Q22 = win_sum(X2[2] * X2[2])
    Q01 = win_sum(X2[0] * X2[1])
    Q02 = win_sum(X2[0] * X2[2])
    Q12 = win_sum(X2[1] * X2[2])

    wn = f32(_WIN)
    a0 = wn / s0
    a1 = wn / s1
    a2 = wn / s2
    M00 = a0 * a0 * Q00 - wn
    M11 = a1 * a1 * Q11 - wn
    M22 = a2 * a2 * Q22 - wn
    M01 = a0 * a1 * Q01 - wn
    M02 = a0 * a2 * Q02 - wn
    M12 = a1 * a2 * Q12 - wn

    w00 = scal_ref[0]
    w01 = scal_ref[1]
    w02 = scal_ref[2]
    w10 = scal_ref[3]
    w11 = scal_ref[4]
    w12 = scal_ref[5]

    A2 = (w00 * w00 * M00 + w01 * w01 * M11 + w02 * w02 * M22
          + 2.0 * (w00 * w01 * M01 + w00 * w02 * M02 + w01 * w02 * M12))
    B2 = (w10 * w10 * M00 + w11 * w11 * M11 + w12 * w12 * M22
          + 2.0 * (w10 * w11 * M01 + w10 * w12 * M02 + w11 * w12 * M12))
    r = jnp.sqrt(jnp.maximum(A2, 0.0) / B2)

    g0 = w00 + r * w10
    g1 = w01 + r * w11
    g2 = w02 + r * w12
    p0 = g0 * a0
    p1 = g1 * a1
    p2 = g2 * a2
    p3 = g0 + g1 + g2

    rr = jax.lax.broadcasted_iota(jnp.int32, (R1, 128), 0)
    cc = jax.lax.broadcasted_iota(jnp.int32, (R1, 128), 1)
    kg = (pid * RB - 1 + rr) * _LANE + cc
    valid = (kg >= 0) & (kg < K)
    p0 = jnp.where(valid, p0, 0.0)
    p1 = jnp.where(valid, p1, 0.0)
    p2 = jnp.where(valid, p2, 0.0)
    p3 = jnp.where(valid, p3, 0.0)

    def back_sum(p):
        pr = jnp.concatenate([p[:RB, :], p[1:R1, :]], axis=1)
        return jnp.dot(pr, T2, preferred_element_type=f32)

    P0 = back_sum(p0)
    P1 = back_sum(p1)
    P2 = back_sum(p2)
    P3 = back_sum(p3)

    out_ref[...] = (ch[0][1:RB + 1, :] * P0 + ch[1][1:RB + 1, :] * P1
                    + ch[2][1:RB + 1, :] * P2 - P3)


def kernel(rgbs, W, b):
    del b
    N = rgbs.shape[1]
    K = N - _WIN
    nrows = N // _LANE
    assert nrows * _LANE == N
    G = -(-nrows // _RB)

    x3 = jax.lax.optimization_barrier(
        jnp.transpose(rgbs[0]).reshape(3, nrows, _LANE))
    zrow = jnp.zeros((3, 1, _LANE), jnp.float32)
    hl = jnp.concatenate([zrow, x3[:, _RB - 1::_RB, :][:, :G - 1, :]], axis=1)
    hr_rows = x3[:, _RB::_RB, :]
    hr = jnp.concatenate(
        [hr_rows, jnp.zeros((3, G - hr_rows.shape[1], _LANE), jnp.float32)],
        axis=1)
    hl = jnp.swapaxes(hl, 0, 1)
    hr = jnp.swapaxes(hr, 0, 1)
    scal = jnp.concatenate([W[0], W[1]]).astype(jnp.float32)

    out = pl.pallas_call(
        functools.partial(_pos_body, K=K, nrows=nrows),
        grid=(G,),
        in_specs=[
            pl.BlockSpec(memory_space=pltpu.SMEM),
            pl.BlockSpec((3, _RB, _LANE), lambda g: (0, g, 0)),
            pl.BlockSpec((1, 3, _LANE), lambda g: (g, 0, 0)),
            pl.BlockSpec((1, 3, _LANE), lambda g: (g, 0, 0)),
        ],
        out_specs=pl.BlockSpec((_RB, _LANE), lambda g: (g, 0)),
        out_shape=jax.ShapeDtypeStruct((nrows, _LANE), jnp.float32),
        compiler_params=pltpu.CompilerParams(
            dimension_semantics=("parallel",)),
    )(scal, x3, hl, hr)
    return out.reshape(1, N)


# speedup vs baseline: 1.2489x; 1.0370x over previous
import functools

import jax
import jax.numpy as jnp
from jax.experimental import pallas as pl
from jax.experimental.pallas import tpu as pltpu

_WIN = 48
_LANE = 128
_RB = 128


def _pos_body(w_ref, x_ref, hl_ref, hr_ref, out_ref, *, K, nrows):
    RB = _RB
    R1 = RB + 1
    f32 = jnp.float32

    hl = hl_ref[...]
    hr = hr_ref[...]

    pid = pl.program_id(0)

    crow = jax.lax.broadcasted_iota(jnp.int32, (RB + 2, 128), 0)
    in_range = (pid * RB - 1 + crow) < nrows
    ch = [jnp.where(in_range,
                    jnp.concatenate([hl[:, c, :], x_ref[c], hr[:, c, :]],
                                    axis=0), 0.0)
          for c in range(3)]

    X2 = [jnp.concatenate([c_[:R1, :], c_[1:R1 + 1, :]], axis=1) for c_ in ch]

    ii = jax.lax.broadcasted_iota(jnp.int32, (256, 128), 0)
    ll = jax.lax.broadcasted_iota(jnp.int32, (256, 128), 1)
    T1 = ((ii >= ll) & (ii <= ll + (_WIN - 1))).astype(f32)
    T2 = ((ii >= ll + (_LANE - _WIN + 1)) & (ii <= ll + _LANE)).astype(f32)

    def win_sum(a):
        return jnp.dot(a, T1, preferred_element_type=f32)

    s0 = win_sum(X2[0])
    s1 = win_sum(X2[1])
    s2 = win_sum(X2[2])
    Q00 = win_sum(X2[0] * X2[0])
    Q11 = win_sum(X2[1] * X2[1])
    Q22 = win_sum(X2[2] * X2[2])
    Q01 = win_sum(X2[0] * X2[1])
    Q02 = win_sum(X2[0] * X2[2])
    Q12 = win_sum(X2[1] * X2[2])

    wn = f32(_WIN)
    a0 = wn / s0
    a1 = wn / s1
    a2 = wn / s2
    M00 = a0 * a0 * Q00 - wn
    M11 = a1 * a1 * Q11 - wn
    M22 = a2 * a2 * Q22 - wn
    M01 = a0 * a1 * Q01 - wn
    M02 = a0 * a2 * Q02 - wn
    M12 = a1 * a2 * Q12 - wn

    w00 = w_ref[0, 0]
    w01 = w_ref[0, 1]
    w02 = w_ref[0, 2]
    w10 = w_ref[1, 0]
    w11 = w_ref[1, 1]
    w12 = w_ref[1, 2]

    A2 = (w00 * w00 * M00 + w01 * w01 * M11 + w02 * w02 * M22
          + 2.0 * (w00 * w01 * M01 + w00 * w02 * M02 + w01 * w02 * M12))
    B2 = (w10 * w10 * M00 + w11 * w11 * M11 + w12 * w12 * M22
          + 2.0 * (w10 * w11 * M01 + w10 * w12 * M02 + w11 * w12 * M12))
    r = jnp.sqrt(jnp.maximum(A2, 0.0) / B2)

    g0 = w00 + r * w10
    g1 = w01 + r * w11
    g2 = w02 + r * w12
    p0 = g0 * a0
    p1 = g1 * a1
    p2 = g2 * a2
    p3 = g0 + g1 + g2

    rr = jax.lax.broadcasted_iota(jnp.int32, (R1, 128), 0)
    cc = jax.lax.broadcasted_iota(jnp.int32, (R1, 128), 1)
    kg = (pid * RB - 1 + rr) * _LANE + cc
    valid = (kg >= 0) & (kg < K)
    p0 = jnp.where(valid, p0, 0.0)
    p1 = jnp.where(valid, p1, 0.0)
    p2 = jnp.where(valid, p2, 0.0)
    p3 = jnp.where(valid, p3, 0.0)

    def back_sum(p):
        pr = jnp.concatenate([p[:RB, :], p[1:R1, :]], axis=1)
        return jnp.dot(pr, T2, preferred_element_type=f32)

    P0 = back_sum(p0)
    P1 = back_sum(p1)
    P2 = back_sum(p2)
    P3 = back_sum(p3)

    out_ref[...] = (ch[0][1:RB + 1, :] * P0 + ch[1][1:RB + 1, :] * P1
                    + ch[2][1:RB + 1, :] * P2 - P3)


def kernel(rgbs, W, b):
    del b
    N = rgbs.shape[1]
    K = N - _WIN
    nrows = N // _LANE
    assert nrows * _LANE == N
    G = -(-nrows // _RB)

    x3 = jax.lax.optimization_barrier(
        jnp.transpose(rgbs[0]).reshape(3, nrows, _LANE))
    zrow = jnp.zeros((3, 1, _LANE), jnp.float32)
    hl = jnp.concatenate([zrow, x3[:, _RB - 1::_RB, :][:, :G - 1, :]], axis=1)
    hr_rows = x3[:, _RB::_RB, :]
    hr = jnp.concatenate(
        [hr_rows, jnp.zeros((3, G - hr_rows.shape[1], _LANE), jnp.float32)],
        axis=1)
    hl = jnp.swapaxes(hl, 0, 1)
    hr = jnp.swapaxes(hr, 0, 1)
    out = pl.pallas_call(
        functools.partial(_pos_body, K=K, nrows=nrows),
        grid=(G,),
        in_specs=[
            pl.BlockSpec(memory_space=pltpu.SMEM),
            pl.BlockSpec((3, _RB, _LANE), lambda g: (0, g, 0)),
            pl.BlockSpec((1, 3, _LANE), lambda g: (g, 0, 0)),
            pl.BlockSpec((1, 3, _LANE), lambda g: (g, 0, 0)),
        ],
        out_specs=pl.BlockSpec((_RB, _LANE), lambda g: (g, 0)),
        out_shape=jax.ShapeDtypeStruct((nrows, _LANE), jnp.float32),
        compiler_params=pltpu.CompilerParams(
            dimension_semantics=("parallel",)),
    )(W.astype(jnp.float32), x3, hl, hr)
    return out.reshape(1, N)


# speedup vs baseline: 1.6017x; 1.2825x over previous
import functools

import jax
import jax.numpy as jnp
from jax.experimental import pallas as pl
from jax.experimental.pallas import tpu as pltpu

_WIN = 48
_LANE = 128
_RB = 320


def _pos_body(w_ref, x_ref, h_ref, out_ref, *, K, nrows):
    RB = _RB
    R1 = RB + 1
    f32 = jnp.float32

    hb = h_ref[0]

    pid = pl.program_id(0)

    crow = jax.lax.broadcasted_iota(jnp.int32, (RB + 2, 128), 0)
    in_range = (pid * RB - 1 + crow) < nrows
    ch = [jnp.where(in_range,
                    jnp.concatenate(
                        [hb[c:c + 1, :], x_ref[c], hb[3 + c:4 + c, :]],
                        axis=0), 0.0)
          for c in range(3)]

    X2 = [jnp.concatenate([c_[:R1, :], c_[1:R1 + 1, :]], axis=1) for c_ in ch]

    ii = jax.lax.broadcasted_iota(jnp.int32, (256, 128), 0)
    ll = jax.lax.broadcasted_iota(jnp.int32, (256, 128), 1)
    T1 = ((ii >= ll) & (ii <= ll + (_WIN - 1))).astype(f32)
    T2 = ((ii >= ll + (_LANE - _WIN + 1)) & (ii <= ll + _LANE)).astype(f32)

    def win_sum(a):
        return jnp.dot(a, T1, preferred_element_type=f32)

    s0 = win_sum(X2[0])
    s1 = win_sum(X2[1])
    s2 = win_sum(X2[2])
    Q00 = win_sum(X2[0] * X2[0])
    Q11 = win_sum(X2[1] * X2[1])
    Q22 = win_sum(X2[2] * X2[2])
    Q01 = win_sum(X2[0] * X2[1])
    Q02 = win_sum(X2[0] * X2[2])
    Q12 = win_sum(X2[1] * X2[2])

    wn = f32(_WIN)
    a0 = wn / s0
    a1 = wn / s1
    a2 = wn / s2
    M00 = a0 * a0 * Q00 - wn
    M11 = a1 * a1 * Q11 - wn
    M22 = a2 * a2 * Q22 - wn
    M01 = a0 * a1 * Q01 - wn
    M02 = a0 * a2 * Q02 - wn
    M12 = a1 * a2 * Q12 - wn

    w00 = w_ref[0, 0]
    w01 = w_ref[0, 1]
    w02 = w_ref[0, 2]
    w10 = w_ref[1, 0]
    w11 = w_ref[1, 1]
    w12 = w_ref[1, 2]

    A2 = (w00 * w00 * M00 + w01 * w01 * M11 + w02 * w02 * M22
          + 2.0 * (w00 * w01 * M01 + w00 * w02 * M02 + w01 * w02 * M12))
    B2 = (w10 * w10 * M00 + w11 * w11 * M11 + w12 * w12 * M22
          + 2.0 * (w10 * w11 * M01 + w10 * w12 * M02 + w11 * w12 * M12))
    r = jnp.sqrt(jnp.maximum(A2, 0.0) / B2)

    g0 = w00 + r * w10
    g1 = w01 + r * w11
    g2 = w02 + r * w12
    p0 = g0 * a0
    p1 = g1 * a1
    p2 = g2 * a2
    p3 = g0 + g1 + g2

    rr = jax.lax.broadcasted_iota(jnp.int32, (R1, 128), 0)
    cc = jax.lax.broadcasted_iota(jnp.int32, (R1, 128), 1)
    kg = (pid * RB - 1 + rr) * _LANE + cc
    valid = (kg >= 0) & (kg < K)
    p0 = jnp.where(valid, p0, 0.0)
    p1 = jnp.where(valid, p1, 0.0)
    p2 = jnp.where(valid, p2, 0.0)
    p3 = jnp.where(valid, p3, 0.0)

    def back_sum(p):
        pr = jnp.concatenate([p[:RB, :], p[1:R1, :]], axis=1)
        return jnp.dot(pr, T2, preferred_element_type=f32)

    P0 = back_sum(p0)
    P1 = back_sum(p1)
    P2 = back_sum(p2)
    P3 = back_sum(p3)

    out_ref[...] = (ch[0][1:RB + 1, :] * P0 + ch[1][1:RB + 1, :] * P1
                    + ch[2][1:RB + 1, :] * P2 - P3)


def kernel(rgbs, W, b):
    del b
    N = rgbs.shape[1]
    K = N - _WIN
    nrows = N // _LANE
    assert nrows * _LANE == N
    G = -(-nrows // _RB)

    x3 = jax.lax.optimization_barrier(
        jnp.transpose(rgbs[0]).reshape(3, nrows, _LANE))
    zrow = jnp.zeros((3, 1, _LANE), jnp.float32)
    hl = jnp.concatenate([zrow, x3[:, _RB - 1::_RB, :][:, :G - 1, :]], axis=1)
    hr_rows = x3[:, _RB::_RB, :]
    hr = jnp.concatenate(
        [hr_rows, jnp.zeros((3, G - hr_rows.shape[1], _LANE), jnp.float32)],
        axis=1)
    h8 = jnp.concatenate(
        [jnp.swapaxes(hl, 0, 1), jnp.swapaxes(hr, 0, 1),
         jnp.zeros((G, 2, _LANE), jnp.float32)], axis=1)
    out = pl.pallas_call(
        functools.partial(_pos_body, K=K, nrows=nrows),
        grid=(G,),
        in_specs=[
            pl.BlockSpec(memory_space=pltpu.SMEM),
            pl.BlockSpec((3, _RB, _LANE), lambda g: (0, g, 0)),
            pl.BlockSpec((1, 8, _LANE), lambda g: (g, 0, 0)),
        ],
        out_specs=pl.BlockSpec((_RB, _LANE), lambda g: (g, 0)),
        out_shape=jax.ShapeDtypeStruct((nrows, _LANE), jnp.float32),
        compiler_params=pltpu.CompilerParams(
            dimension_semantics=("parallel",)),
    )(W.astype(jnp.float32), x3, h8)
    return out.reshape(1, N)


# speedup vs baseline: 1.6421x; 1.0252x over previous
import functools

import jax
import jax.numpy as jnp
from jax.experimental import pallas as pl
from jax.experimental.pallas import tpu as pltpu

_WIN = 48
_LANE = 128
_RB = 400


def _pos_body(w_ref, x_ref, h_ref, out_ref, *, K, nrows):
    RB = _RB
    R1 = RB + 1
    f32 = jnp.float32

    hb = h_ref[0]

    pid = pl.program_id(0)

    crow = jax.lax.broadcasted_iota(jnp.int32, (RB + 2, 128), 0)
    in_range = (pid * RB - 1 + crow) < nrows
    ch = [jnp.where(in_range,
                    jnp.concatenate(
                        [hb[c:c + 1, :], x_ref[c], hb[3 + c:4 + c, :]],
                        axis=0), 0.0)
          for c in range(3)]

    X2 = [jnp.concatenate([c_[:R1, :], c_[1:R1 + 1, :]], axis=1) for c_ in ch]

    ii = jax.lax.broadcasted_iota(jnp.int32, (256, 128), 0)
    ll = jax.lax.broadcasted_iota(jnp.int32, (256, 128), 1)
    T1 = ((ii >= ll) & (ii <= ll + (_WIN - 1))).astype(f32)
    T2 = ((ii >= ll + (_LANE - _WIN + 1)) & (ii <= ll + _LANE)).astype(f32)

    def win_sum(a):
        return jnp.dot(a, T1, preferred_element_type=f32)

    s0 = win_sum(X2[0])
    s1 = win_sum(X2[1])
    s2 = win_sum(X2[2])
    Q00 = win_sum(X2[0] * X2[0])
    Q11 = win_sum(X2[1] * X2[1])
    Q22 = win_sum(X2[2] * X2[2])
    Q01 = win_sum(X2[0] * X2[1])
    Q02 = win_sum(X2[0] * X2[2])
    Q12 = win_sum(X2[1] * X2[2])

    wn = f32(_WIN)
    a0 = wn / s0
    a1 = wn / s1
    a2 = wn / s2
    M00 = a0 * a0 * Q00 - wn
    M11 = a1 * a1 * Q11 - wn
    M22 = a2 * a2 * Q22 - wn
    M01 = a0 * a1 * Q01 - wn
    M02 = a0 * a2 * Q02 - wn
    M12 = a1 * a2 * Q12 - wn

    w00 = w_ref[0, 0]
    w01 = w_ref[0, 1]
    w02 = w_ref[0, 2]
    w10 = w_ref[1, 0]
    w11 = w_ref[1, 1]
    w12 = w_ref[1, 2]

    A2 = (w00 * w00 * M00 + w01 * w01 * M11 + w02 * w02 * M22
          + 2.0 * (w00 * w01 * M01 + w00 * w02 * M02 + w01 * w02 * M12))
    B2 = (w10 * w10 * M00 + w11 * w11 * M11 + w12 * w12 * M22
          + 2.0 * (w10 * w11 * M01 + w10 * w12 * M02 + w11 * w12 * M12))
    r = jnp.sqrt(jnp.maximum(A2, 0.0) / B2)

    g0 = w00 + r * w10
    g1 = w01 + r * w11
    g2 = w02 + r * w12
    p0 = g0 * a0
    p1 = g1 * a1
    p2 = g2 * a2
    p3 = g0 + g1 + g2

    rr = jax.lax.broadcasted_iota(jnp.int32, (R1, 128), 0)
    cc = jax.lax.broadcasted_iota(jnp.int32, (R1, 128), 1)
    kg = (pid * RB - 1 + rr) * _LANE + cc
    valid = (kg >= 0) & (kg < K)
    p0 = jnp.where(valid, p0, 0.0)
    p1 = jnp.where(valid, p1, 0.0)
    p2 = jnp.where(valid, p2, 0.0)
    p3 = jnp.where(valid, p3, 0.0)

    def back_sum(p):
        pr = jnp.concatenate([p[:RB, :], p[1:R1, :]], axis=1)
        return jnp.dot(pr, T2, preferred_element_type=f32)

    P0 = back_sum(p0)
    P1 = back_sum(p1)
    P2 = back_sum(p2)
    P3 = back_sum(p3)

    out_ref[...] = (ch[0][1:RB + 1, :] * P0 + ch[1][1:RB + 1, :] * P1
                    + ch[2][1:RB + 1, :] * P2 - P3)


def kernel(rgbs, W, b):
    del b
    N = rgbs.shape[1]
    K = N - _WIN
    nrows = N // _LANE
    assert nrows * _LANE == N
    G = -(-nrows // _RB)

    x3 = jax.lax.optimization_barrier(
        jnp.transpose(rgbs[0]).reshape(3, nrows, _LANE))
    zrow = jnp.zeros((3, 1, _LANE), jnp.float32)
    hl = jnp.concatenate([zrow, x3[:, _RB - 1::_RB, :][:, :G - 1, :]], axis=1)
    hr_rows = x3[:, _RB::_RB, :]
    hr = jnp.concatenate(
        [hr_rows, jnp.zeros((3, G - hr_rows.shape[1], _LANE), jnp.float32)],
        axis=1)
    h8 = jnp.concatenate(
        [jnp.swapaxes(hl, 0, 1), jnp.swapaxes(hr, 0, 1),
         jnp.zeros((G, 2, _LANE), jnp.float32)], axis=1)
    out = pl.pallas_call(
        functools.partial(_pos_body, K=K, nrows=nrows),
        grid=(G,),
        in_specs=[
            pl.BlockSpec(memory_space=pltpu.SMEM),
            pl.BlockSpec((3, _RB, _LANE), lambda g: (0, g, 0)),
            pl.BlockSpec((1, 8, _LANE), lambda g: (g, 0, 0)),
        ],
        out_specs=pl.BlockSpec((_RB, _LANE), lambda g: (g, 0)),
        out_shape=jax.ShapeDtypeStruct((nrows, _LANE), jnp.float32),
        compiler_params=pltpu.CompilerParams(
            dimension_semantics=("parallel",)),
    )(W.astype(jnp.float32), x3, h8)
    return out.reshape(1, N)


# speedup vs baseline: 1.6850x; 1.0261x over previous
import functools

import jax
import jax.numpy as jnp
from jax.experimental import pallas as pl
from jax.experimental.pallas import tpu as pltpu

_WIN = 48
_LANE = 128
_RB = 800


def _pos_body(w_ref, x_ref, h_ref, out_ref, *, K, nrows):
    RB = _RB
    R1 = RB + 1
    f32 = jnp.float32

    hb = h_ref[0]

    pid = pl.program_id(0)

    crow = jax.lax.broadcasted_iota(jnp.int32, (RB + 2, 128), 0)
    in_range = (pid * RB - 1 + crow) < nrows
    ch = [jnp.where(in_range,
                    jnp.concatenate(
                        [hb[c:c + 1, :], x_ref[c], hb[3 + c:4 + c, :]],
                        axis=0), 0.0)
          for c in range(3)]

    X2 = [jnp.concatenate([c_[:R1, :], c_[1:R1 + 1, :]], axis=1) for c_ in ch]

    ii = jax.lax.broadcasted_iota(jnp.int32, (256, 128), 0)
    ll = jax.lax.broadcasted_iota(jnp.int32, (256, 128), 1)
    T1 = ((ii >= ll) & (ii <= ll + (_WIN - 1))).astype(f32)
    T2 = ((ii >= ll + (_LANE - _WIN + 1)) & (ii <= ll + _LANE)).astype(f32)

    def win_sum(a):
        return jnp.dot(a, T1, preferred_element_type=f32)

    s0 = win_sum(X2[0])
    s1 = win_sum(X2[1])
    s2 = win_sum(X2[2])
    Q00 = win_sum(X2[0] * X2[0])
    Q11 = win_sum(X2[1] * X2[1])
    Q22 = win_sum(X2[2] * X2[2])
    Q01 = win_sum(X2[0] * X2[1])
    Q02 = win_sum(X2[0] * X2[2])
    Q12 = win_sum(X2[1] * X2[2])

    wn = f32(_WIN)
    a0 = wn / s0
    a1 = wn / s1
    a2 = wn / s2
    M00 = a0 * a0 * Q00 - wn
    M11 = a1 * a1 * Q11 - wn
    M22 = a2 * a2 * Q22 - wn
    M01 = a0 * a1 * Q01 - wn
    M02 = a0 * a2 * Q02 - wn
    M12 = a1 * a2 * Q12 - wn

    w00 = w_ref[0, 0]
    w01 = w_ref[0, 1]
    w02 = w_ref[0, 2]
    w10 = w_ref[1, 0]
    w11 = w_ref[1, 1]
    w12 = w_ref[1, 2]

    A2 = (w00 * w00 * M00 + w01 * w01 * M11 + w02 * w02 * M22
          + 2.0 * (w00 * w01 * M01 + w00 * w02 * M02 + w01 * w02 * M12))
    B2 = (w10 * w10 * M00 + w11 * w11 * M11 + w12 * w12 * M22
          + 2.0 * (w10 * w11 * M01 + w10 * w12 * M02 + w11 * w12 * M12))
    r = jnp.sqrt(jnp.maximum(A2, 0.0) / B2)

    g0 = w00 + r * w10
    g1 = w01 + r * w11
    g2 = w02 + r * w12
    p0 = g0 * a0
    p1 = g1 * a1
    p2 = g2 * a2
    p3 = g0 + g1 + g2

    rr = jax.lax.broadcasted_iota(jnp.int32, (R1, 128), 0)
    cc = jax.lax.broadcasted_iota(jnp.int32, (R1, 128), 1)
    kg = (pid * RB - 1 + rr) * _LANE + cc
    valid = (kg >= 0) & (kg < K)
    p0 = jnp.where(valid, p0, 0.0)
    p1 = jnp.where(valid, p1, 0.0)
    p2 = jnp.where(valid, p2, 0.0)
    p3 = jnp.where(valid, p3, 0.0)

    def back_sum(p):
        pr = jnp.concatenate([p[:RB, :], p[1:R1, :]], axis=1)
        return jnp.dot(pr, T2, preferred_element_type=f32)

    P0 = back_sum(p0)
    P1 = back_sum(p1)
    P2 = back_sum(p2)
    P3 = back_sum(p3)

    out_ref[...] = (ch[0][1:RB + 1, :] * P0 + ch[1][1:RB + 1, :] * P1
                    + ch[2][1:RB + 1, :] * P2 - P3)


def kernel(rgbs, W, b):
    del b
    N = rgbs.shape[1]
    K = N - _WIN
    nrows = N // _LANE
    assert nrows * _LANE == N
    G = -(-nrows // _RB)

    x3 = jax.lax.optimization_barrier(
        jnp.transpose(rgbs[0]).reshape(3, nrows, _LANE))
    zrow = jnp.zeros((3, 1, _LANE), jnp.float32)
    hl = jnp.concatenate([zrow, x3[:, _RB - 1::_RB, :][:, :G - 1, :]], axis=1)
    hr_rows = x3[:, _RB::_RB, :]
    hr = jnp.concatenate(
        [hr_rows, jnp.zeros((3, G - hr_rows.shape[1], _LANE), jnp.float32)],
        axis=1)
    h8 = jnp.concatenate(
        [jnp.swapaxes(hl, 0, 1), jnp.swapaxes(hr, 0, 1),
         jnp.zeros((G, 2, _LANE), jnp.float32)], axis=1)
    out = pl.pallas_call(
        functools.partial(_pos_body, K=K, nrows=nrows),
        grid=(G,),
        in_specs=[
            pl.BlockSpec(memory_space=pltpu.SMEM),
            pl.BlockSpec((3, _RB, _LANE), lambda g: (0, g, 0)),
            pl.BlockSpec((1, 8, _LANE), lambda g: (g, 0, 0)),
        ],
        out_specs=pl.BlockSpec((_RB, _LANE), lambda g: (g, 0)),
        out_shape=jax.ShapeDtypeStruct((nrows, _LANE), jnp.float32),
        compiler_params=pltpu.CompilerParams(
            dimension_semantics=("parallel",)),
    )(W.astype(jnp.float32), x3, h8)
    return out.reshape(1, N)
